# Initial kernel scaffold; baseline (speedup 1.0000x reference)
import jax
import jax.numpy as jnp
from jax.experimental import pallas as pl


def kernel(pT_bins, eta_bins, phi_bins, counts, pT_table, eta_table, phi_table, start_token, stop_token, ln_gamma, ln_beta):
    raise NotImplementedError("write your pallas kernel here")


# speedup vs baseline: 7.4162x; 7.4162x over previous
import functools

import jax
import jax.numpy as jnp
from jax.experimental import pallas as pl
from jax.experimental.pallas import tpu as pltpu

B = 1024
N = 100
D = 512
S = N + 2
PT_SLOTS = 42
ETA_SLOTS = 32
PHI_SLOTS = 32
C = PT_SLOTS + ETA_SLOTS + PHI_SLOTS
GB = 8


def _embed_body(pt_ref, eta_ref, phi_ref, cnt_ref, tab_ref, start_ref,
                stop_ref, gamma_ref, beta_ref, out_ref):
    r = GB * N
    idx_pt = jnp.clip(pt_ref[...].reshape(r, 1) + 1, 0, PT_SLOTS - 1)
    idx_eta = jnp.clip(eta_ref[...].reshape(r, 1) + 1, 0, ETA_SLOTS - 1) + PT_SLOTS
    idx_phi = (jnp.clip(phi_ref[...].reshape(r, 1) + 1, 0, PHI_SLOTS - 1)
               + PT_SLOTS + ETA_SLOTS)

    col = jax.lax.broadcasted_iota(jnp.int32, (r, C), 1)
    oh = ((idx_pt == col).astype(jnp.float32)
          + (idx_eta == col).astype(jnp.float32)
          + (idx_phi == col).astype(jnp.float32))
    e = jax.lax.dot_general(oh, tab_ref[...], (((1,), (0,)), ((), ())),
                            preferred_element_type=jnp.float32)
    e = e.reshape(GB, N, D)

    z1 = jnp.zeros((GB, 1, D), jnp.float32)
    e1 = jnp.concatenate([z1, e, z1], axis=1)
    e2 = jnp.concatenate([z1, z1, e], axis=1)

    s_vec = jax.lax.broadcasted_iota(jnp.int32, (GB, S, 1), 1)
    cnt3 = cnt_ref[...].reshape(GB, 1, 1)
    take1 = (s_vec >= 1) & (s_vec <= cnt3)
    take2 = s_vec >= cnt3 + 2
    is_start = s_vec == 0
    is_stop = (s_vec == cnt3 + 1) & (cnt3 < N)

    rows = jnp.where(take1, e1, 0.0) + jnp.where(take2, e2, 0.0)
    rows = rows + is_start.astype(jnp.float32) * start_ref[...].reshape(1, 1, D)
    rows = rows + is_stop.astype(jnp.float32) * stop_ref[...].reshape(1, 1, D)

    mean = jnp.mean(rows, axis=-1, keepdims=True)
    xc = rows - mean
    var = jnp.mean(xc * xc, axis=-1, keepdims=True)
    out = xc * jax.lax.rsqrt(var + 1e-5)
    out_ref[...] = out * gamma_ref[...].reshape(1, 1, D) + beta_ref[...].reshape(1, 1, D)


@functools.partial(jax.jit, static_argnames=())
def kernel(pT_bins, eta_bins, phi_bins, counts, pT_table, eta_table,
           phi_table, start_token, stop_token, ln_gamma, ln_beta):
    tab = jnp.concatenate([pT_table.at[0].set(0.0),
                           eta_table.at[0].set(0.0),
                           phi_table.at[0].set(0.0)], axis=0)
    counts2 = counts.reshape(B, 1).astype(jnp.int32)
    grid = (B // GB,)
    out = pl.pallas_call(
        _embed_body,
        grid=grid,
        in_specs=[
            pl.BlockSpec((GB, N, 1), lambda i: (i, 0, 0)),
            pl.BlockSpec((GB, N, 1), lambda i: (i, 0, 0)),
            pl.BlockSpec((GB, N, 1), lambda i: (i, 0, 0)),
            pl.BlockSpec((GB, 1), lambda i: (i, 0)),
            pl.BlockSpec((C, D), lambda i: (0, 0)),
            pl.BlockSpec((1, D), lambda i: (0, 0)),
            pl.BlockSpec((1, D), lambda i: (0, 0)),
            pl.BlockSpec((1, D), lambda i: (0, 0)),
            pl.BlockSpec((1, D), lambda i: (0, 0)),
        ],
        out_specs=pl.BlockSpec((GB, S, D), lambda i: (i, 0, 0)),
        out_shape=jax.ShapeDtypeStruct((B, S, D), jnp.float32),
        compiler_params=pltpu.CompilerParams(
            dimension_semantics=("parallel",)),
    )(pT_bins.astype(jnp.int32).reshape(B, N, 1),
      eta_bins.astype(jnp.int32).reshape(B, N, 1),
      phi_bins.astype(jnp.int32).reshape(B, N, 1), counts2, tab,
      start_token, stop_token,
      ln_gamma.reshape(1, D), ln_beta.reshape(1, D))
    return out
